# Initial kernel scaffold; baseline (speedup 1.0000x reference)
import jax
import jax.numpy as jnp
from jax.experimental import pallas as pl


def kernel(scores, class_indices):
    raise NotImplementedError("write your pallas kernel here")


# speedup vs baseline: 2.1711x; 2.1711x over previous
import functools
import math

import jax
import jax.numpy as jnp
from jax import lax
from jax.experimental import pallas as pl
from jax.experimental.pallas import tpu as pltpu
from jax.experimental.pallas import tpu_sc as plsc

_BATCH = 1024
_VOCAB = 65536
_CODE_LEN = 16
_LANES = 16
_NUM_WORKERS = 32
_ROWS_PER_W = _BATCH // _NUM_WORKERS
_GROUPS = _ROWS_PER_W // _LANES


def _sc_body(scores_hbm, cls_hbm, probs_hbm, cls_v, vals_v, probs_v, sem):
    wid = lax.axis_index("s") * 2 + lax.axis_index("c")
    base = wid * _ROWS_PER_W

    pltpu.sync_copy(cls_hbm.at[pl.ds(base, _ROWS_PER_W)], cls_v)
    iota = lax.iota(jnp.int32, _LANES)

    copies = []
    for g in range(_GROUPS):
        c = cls_v[pl.ds(g * _LANES, _LANES)]
        rowbase = (base + g * _LANES + iota) * _VOCAB
        prefix = jnp.zeros((_LANES,), jnp.int32)
        for k in range(_CODE_LEN):
            bit = (c >> (_CODE_LEN - 1 - k)) & 1
            idx = rowbase + ((1 << k) - 1) + prefix
            cp = pltpu.make_async_copy(
                scores_hbm.at[idx], vals_v.at[g * _CODE_LEN + k], sem
            )
            cp.start()
            copies.append(cp)
            prefix = prefix + bit
    for cp in copies:
        cp.wait()

    one = jnp.float32(1.0)
    for g in range(_GROUPS):
        c = cls_v[pl.ds(g * _LANES, _LANES)]
        acc = jnp.ones((_LANES,), jnp.float32)
        for k in range(_CODE_LEN):
            bit = (c >> (_CODE_LEN - 1 - k)) & 1
            s = vals_v[g * _CODE_LEN + k, :]
            z = jnp.where(bit == 1, -s, s)
            e = jnp.exp(-jnp.abs(z))
            num = jnp.where(z >= 0, one, e)
            acc = acc * (num / (one + e))
        probs_v[pl.ds(g * _LANES, _LANES)] = acc

    pltpu.sync_copy(probs_v, probs_hbm.at[pl.ds(base, _ROWS_PER_W)])


@functools.cache
def _sc_probs():
    return pl.kernel(
        _sc_body,
        mesh=plsc.VectorSubcoreMesh(core_axis_name="c", subcore_axis_name="s"),
        out_type=jax.ShapeDtypeStruct((_BATCH,), jnp.float32),
        scratch_types=[
            pltpu.VMEM((_ROWS_PER_W,), jnp.int32),
            pltpu.VMEM((_GROUPS * _CODE_LEN, _LANES), jnp.float32),
            pltpu.VMEM((_ROWS_PER_W,), jnp.float32),
            pltpu.SemaphoreType.DMA,
        ],
    )


def _loss_body(p_ref, out_ref):
    out_ref[0, 0] = jnp.sum(-jnp.log(p_ref[:])) * jnp.float32(1.0 / _BATCH)


_tc_loss = pl.pallas_call(
    _loss_body,
    out_shape=jax.ShapeDtypeStruct((1, 1), jnp.float32),
    out_specs=pl.BlockSpec(memory_space=pltpu.SMEM),
)


def kernel(scores, class_indices):
    probs = _sc_probs()(scores.reshape(-1), class_indices)
    loss = _tc_loss(probs.reshape(8, 128))
    return loss[0, 0]


# speedup vs baseline: 17.5513x; 8.0841x over previous
import functools
import math

import jax
import jax.numpy as jnp
from jax import lax
from jax.experimental import pallas as pl
from jax.experimental.pallas import tpu as pltpu
from jax.experimental.pallas import tpu_sc as plsc

_BATCH = 1024
_VOCAB = 65536
_CODE_LEN = 16
_LANES = 16
_NUM_WORKERS = 32
_ROWS_PER_W = _BATCH // _NUM_WORKERS
_GROUPS = _ROWS_PER_W // _LANES

_BLOCKS = sorted(
    {((1 << k) - 1) >> 7 for k in range(_CODE_LEN)}
    | {((1 << k) - 1 + k) >> 7 for k in range(_CODE_LEN)}
)
_SLOT = {blk: i for i, blk in enumerate(_BLOCKS)}
_NUM_BLOCKS = len(_BLOCKS)


def _sc_body(scores_hbm, cls_hbm, probs_hbm, cls_v, vals_v, probs_v, sem):
    wid = lax.axis_index("s") * 2 + lax.axis_index("c")
    base = wid * _ROWS_PER_W

    pltpu.sync_copy(cls_hbm.at[pl.ds(base, _ROWS_PER_W)], cls_v)
    iota = lax.iota(jnp.int32, _LANES)

    copies = []
    for slot, blk in enumerate(_BLOCKS):
        cp = pltpu.make_async_copy(
            scores_hbm.at[pl.ds(base, _ROWS_PER_W), pl.ds(blk * 128, 128)],
            vals_v.at[slot],
            sem,
        )
        cp.start()
        copies.append(cp)
    for cp in copies:
        cp.wait()

    one = jnp.float32(1.0)
    for g in range(_GROUPS):
        c = cls_v[pl.ds(g * _LANES, _LANES)]
        rows = g * _LANES + iota
        acc = jnp.ones((_LANES,), jnp.float32)
        prefix = jnp.zeros((_LANES,), jnp.int32)
        for k in range(_CODE_LEN):
            bit = (c >> (_CODE_LEN - 1 - k)) & 1
            col = ((1 << k) - 1) + prefix
            lo_blk = ((1 << k) - 1) >> 7
            slot = _SLOT[lo_blk] + ((col >> 7) - lo_blk)
            s = plsc.load_gather(vals_v, [slot, rows, col & 127])
            z = jnp.where(bit == 1, -s, s)
            e = jnp.exp(-jnp.abs(z))
            num = jnp.where(z >= 0, one, e)
            acc = acc * (num / (one + e))
            prefix = prefix + bit
        probs_v[pl.ds(g * _LANES, _LANES)] = acc

    pltpu.sync_copy(probs_v, probs_hbm.at[pl.ds(base, _ROWS_PER_W)])


@functools.cache
def _sc_probs():
    return pl.kernel(
        _sc_body,
        mesh=plsc.VectorSubcoreMesh(core_axis_name="c", subcore_axis_name="s"),
        out_type=jax.ShapeDtypeStruct((_BATCH,), jnp.float32),
        compiler_params=pltpu.CompilerParams(needs_layout_passes=False),
        scratch_types=[
            pltpu.VMEM((_ROWS_PER_W,), jnp.int32),
            pltpu.VMEM((_NUM_BLOCKS, _ROWS_PER_W, 128), jnp.float32),
            pltpu.VMEM((_ROWS_PER_W,), jnp.float32),
            pltpu.SemaphoreType.DMA,
        ],
    )


def _loss_body(p_ref, out_ref):
    out_ref[0, 0] = jnp.sum(-jnp.log(p_ref[:])) * jnp.float32(1.0 / _BATCH)


_tc_loss = pl.pallas_call(
    _loss_body,
    out_shape=jax.ShapeDtypeStruct((1, 1), jnp.float32),
    out_specs=pl.BlockSpec(memory_space=pltpu.SMEM),
)


def kernel(scores, class_indices):
    probs = _sc_probs()(scores, class_indices)
    loss = _tc_loss(probs.reshape(8, 128))
    return loss[0, 0]
